# Initial kernel scaffold; baseline (speedup 1.0000x reference)
import jax
import jax.numpy as jnp
from jax.experimental import pallas as pl


def kernel(x, label, centers):
    raise NotImplementedError("write your pallas kernel here")


# speedup vs baseline: 7.3284x; 7.3284x over previous
import functools

import jax
import jax.numpy as jnp
from jax import lax
from jax.experimental import pallas as pl
from jax.experimental.pallas import tpu as pltpu
from jax.experimental.pallas import tpu_sc as plsc

CLASS_NUM = 100000
PART_NUM = 8
FEA_DIM = 32
BATCH = 16384
LAMBDA_C = 1.0
ROW = PART_NUM * FEA_DIM

NC = 2
NS = 16
NW = NC * NS
BPW = BATCH // NW
CH = 64
NCHUNK = BPW // CH
LANES = 16
VPR = ROW // LANES

_sc_mesh = plsc.VectorSubcoreMesh(core_axis_name="c", subcore_axis_name="s")


@functools.partial(
    pl.kernel,
    out_type=jax.ShapeDtypeStruct((NW, LANES), jnp.float32),
    mesh=_sc_mesh,
    scratch_types=[
        pltpu.VMEM((BPW,), jnp.int32),
        pltpu.VMEM((CH, ROW), jnp.float32),
        pltpu.VMEM((CH, ROW), jnp.float32),
        pltpu.VMEM((CH, ROW), jnp.float32),
        pltpu.VMEM((CH, ROW), jnp.float32),
        pltpu.VMEM((LANES,), jnp.float32),
        pltpu.SemaphoreType.DMA,
        pltpu.SemaphoreType.DMA,
        pltpu.SemaphoreType.DMA,
        pltpu.SemaphoreType.DMA,
    ],
)
def _loss_partials(x_hbm, lbl_hbm, centers_hbm, out_hbm,
                   idx_v, gb0, gb1, xb0, xb1, acc_v,
                   sg0, sg1, sx0, sx1):
    wid = lax.axis_index("s") * NC + lax.axis_index("c")
    base = wid * BPW
    pltpu.sync_copy(lbl_hbm.at[pl.ds(base, BPW)], idx_v)

    gbufs = (gb0, gb1)
    xbufs = (xb0, xb1)
    gsems = (sg0, sg1)
    xsems = (sx0, sx1)

    def start(c):
        slot = c % 2
        cbase = c * CH
        g = pltpu.async_copy(
            centers_hbm.at[idx_v.at[pl.ds(cbase, CH)]], gbufs[slot], gsems[slot])
        x = pltpu.async_copy(
            x_hbm.at[pl.ds(base + cbase, CH)], xbufs[slot], xsems[slot])
        return g, x

    acc = jnp.zeros((LANES,), jnp.float32)
    pend = start(0)
    for c in range(NCHUNK):
        nxt = start(c + 1) if c + 1 < NCHUNK else None
        gcopy, xcopy = pend
        gcopy.wait()
        xcopy.wait()
        gb = gbufs[c % 2]
        xb = xbufs[c % 2]

        def row_body(r, a, gb=gb, xb=xb):
            for v in range(VPR):
                xv = xb[r, pl.ds(v * LANES, LANES)]
                gv = gb[r, pl.ds(v * LANES, LANES)]
                d = xv - gv
                a = a + d * d
            return a

        acc = lax.fori_loop(0, CH, row_body, acc)
        pend = nxt

    acc_v[...] = acc
    pltpu.sync_copy(acc_v, out_hbm.at[wid])


BC = 2000
GRID = CLASS_NUM // BC


def _stats_body(c_ref, g_ref, s1_ref, s2_ref):
    blk = c_ref[...]
    g = lax.dot_general(blk, g_ref[...], (((1,), (0,)), ((), ())),
                        preferred_element_type=jnp.float32)
    i = pl.program_id(0)
    s1_ref[i, 0] = jnp.sum(blk)
    s2_ref[i, 0] = jnp.sum(g * g)


_stats_call = pl.pallas_call(
    _stats_body,
    grid=(GRID,),
    in_specs=[
        pl.BlockSpec((BC, ROW), lambda i: (i, 0)),
        pl.BlockSpec((ROW, PART_NUM), lambda i: (0, 0)),
    ],
    out_specs=[
        pl.BlockSpec((GRID, 1), lambda i: (0, 0), memory_space=pltpu.SMEM),
        pl.BlockSpec((GRID, 1), lambda i: (0, 0), memory_space=pltpu.SMEM),
    ],
    out_shape=[
        jax.ShapeDtypeStruct((GRID, 1), jnp.float32),
        jax.ShapeDtypeStruct((GRID, 1), jnp.float32),
    ],
)


def kernel(x, label, centers):
    lbl = label.astype(jnp.int32)
    x2 = x.reshape(BATCH, ROW)
    c2 = centers.reshape(CLASS_NUM, ROW)

    partials = _loss_partials(x2, lbl, c2)
    gmat = (jnp.arange(ROW, dtype=jnp.int32)[:, None] // FEA_DIM
            == jnp.arange(PART_NUM, dtype=jnp.int32)[None, :]
            ).astype(jnp.float32)
    s1p, s2p = _stats_call(c2, gmat)

    n_all = CLASS_NUM * PART_NUM * FEA_DIM
    s1 = jnp.sum(s1p)
    s2 = jnp.sum(s2p)
    center_mean = s1 / n_all
    mean_m2 = s2 / (CLASS_NUM * PART_NUM * FEA_DIM * FEA_DIM)
    center_var = mean_m2 - center_mean * center_mean
    center_loss = LAMBDA_C * jnp.sum(partials) / (BATCH * PART_NUM * FEA_DIM)
    return (x, center_loss, center_mean, center_var)
